# baseline (speedup 1.0000x reference)
import functools

import jax
import jax.numpy as jnp
from jax import lax
from jax.experimental import pallas as pl
from jax.experimental.pallas import tpu as pltpu
from jax.experimental.pallas import tpu_sc as plsc

_N = 10000
_F_IN = 3703
_H1, _C1 = 4, 16
_H2, _C2 = 1, 6

_NC, _NS = 2, 16
_NW = _NC * _NS
_CH = 128
_NP = 10240


def _mm_body(a_ref, b_ref, o_ref):
    o_ref[...] = jnp.dot(a_ref[...], b_ref[...],
                         preferred_element_type=jnp.float32)


def _matmul(a, b, bm):
    m, k = a.shape
    _, n = b.shape
    return pl.pallas_call(
        _mm_body,
        grid=(m // bm,),
        in_specs=[
            pl.BlockSpec((bm, k), lambda i: (i, 0)),
            pl.BlockSpec((k, n), lambda i: (0, 0)),
        ],
        out_specs=pl.BlockSpec((bm, n), lambda i: (i, 0)),
        out_shape=jax.ShapeDtypeStruct((m, n), jnp.float32),
    )(a, b)


def _comb_body(num_ref, den_ref, b_ref, w_ref, o_ref):
    h = jnp.maximum(num_ref[...] / den_ref[...] + b_ref[...], 0.0)
    o_ref[...] = jnp.dot(h, w_ref[...], preferred_element_type=jnp.float32)


def _combine1(num, den, b, w, bm):
    m, k = num.shape
    n = w.shape[1]
    return pl.pallas_call(
        _comb_body,
        grid=(m // bm,),
        in_specs=[
            pl.BlockSpec((bm, k), lambda i: (i, 0)),
            pl.BlockSpec((bm, k), lambda i: (i, 0)),
            pl.BlockSpec((1, k), lambda i: (0, 0)),
            pl.BlockSpec((k, n), lambda i: (0, 0)),
        ],
        out_specs=pl.BlockSpec((bm, n), lambda i: (i, 0)),
        out_shape=jax.ShapeDtypeStruct((m, n), jnp.float32),
    )(num, den, b, w)


def _final_body(num_ref, den_ref, b_ref, o_ref):
    z = num_ref[...] / den_ref[...] + b_ref[...]
    col = lax.broadcasted_iota(jnp.int32, z.shape, 1)
    zm = jnp.where(col < _H2 * _C2, z, -jnp.inf)
    m = jnp.max(zm, axis=1, keepdims=True)
    s = jnp.sum(jnp.where(col < _H2 * _C2, jnp.exp(z - m), 0.0),
                axis=1, keepdims=True)
    o_ref[...] = (z - m - jnp.log(s))[:, : _H2 * _C2]


def _final(num, den, b, bm):
    m, k = num.shape
    return pl.pallas_call(
        _final_body,
        grid=(m // bm,),
        in_specs=[
            pl.BlockSpec((bm, k), lambda i: (i, 0)),
            pl.BlockSpec((bm, k), lambda i: (i, 0)),
            pl.BlockSpec((1, k), lambda i: (0, 0)),
        ],
        out_specs=pl.BlockSpec((bm, _H2 * _C2), lambda i: (i, 0)),
        out_shape=jax.ShapeDtypeStruct((m, _H2 * _C2), jnp.float32),
    )(num, den, b)


def _splat(v, j):
    return v.at[jnp.full((16,), j, jnp.int32)].get(mode="promise_in_bounds")


def _logits(ad_v, sva, dva, H, h, valid):
    a_s = plsc.load_gather(ad_v, [sva + h])
    a_d = plsc.load_gather(ad_v, [dva + (H + h)])
    al = a_s + a_d
    al = jnp.where(al >= 0, al, al * 0.2)
    return jnp.where(valid, jnp.exp(al), 0.0)


def _edge_body1(e_real, nch, n,
                xw_hbm, ad_hbm, src_hbm, dst_hbm, out_hbm,
                ad_v, src_v, dst_v, rows_v, zer_v,
                sem0, sem1, ssem0, ssem1, out_sh):
    H = _H1
    c = lax.axis_index("c")
    s = lax.axis_index("s")
    wid = s * _NC + c
    rows_per = n // _NS
    r0 = s * rows_per
    aw = 2 * H
    sems = (sem0, sem1)
    ssems = (ssem0, ssem1)

    pltpu.sync_copy(ad_hbm, ad_v)
    pltpu.sync_copy(src_hbm.at[wid], src_v)
    pltpu.sync_copy(dst_hbm.at[wid], dst_v)

    def zrow(r, carry):
        zer_v[r, pl.ds(0, 16)] = jnp.zeros((16,), jnp.float32)
        return carry
    lax.fori_loop(0, _CH, zrow, 0)
    iot = lax.iota(jnp.int32, 16)

    for ph in range(H + 1):
        for j in range(rows_per // _CH):
            pltpu.sync_copy(zer_v, out_sh.at[pl.ds(r0 + j * _CH, _CH)])
        plsc.subcore_barrier()

        def gstart(b, g):
            pltpu.async_copy(xw_hbm.at[ph].at[src_v.at[g]],
                             rows_v.at[b], sems[b])

        def gwait(b):
            pltpu.make_async_copy(xw_hbm.at[ph].at[pl.ds(0, _CH)],
                                  rows_v.at[b], sems[b]).wait()

        def sstart(b, g):
            pltpu.async_copy(rows_v.at[b], out_sh.at[dst_v.at[g]],
                             ssems[b], add=True)

        def swait(b):
            pltpu.make_async_copy(rows_v.at[b], out_sh.at[pl.ds(0, _CH)],
                                  ssems[b]).wait()

        def process(b, g):
            base = (wid * nch + g) * _CH

            def grp(i, carry2):
                sv = src_v[g, pl.ds(i * 16, 16)]
                dv = dst_v[g, pl.ds(i * 16, 16)]
                valid = (base + i * 16 + iot) < e_real
                sva = sv * aw
                dva = dv * aw
                if ph < H:
                    e = _logits(ad_v, sva, dva, H, ph, valid)
                    for k in range(16):
                        row = i * 16 + k
                        v = rows_v[b, row, pl.ds(0, 16)]
                        rows_v[b, row, pl.ds(0, 16)] = v * _splat(e, k)
                else:
                    eh = [_logits(ad_v, sva, dva, H, h, valid)
                          for h in range(H)]
                    for k in range(16):
                        row = i * 16 + k
                        es = _splat(eh[0], k)
                        for h in range(1, H):
                            es = jnp.where(iot == h, _splat(eh[h], k), es)
                        rows_v[b, row, pl.ds(0, 16)] = jnp.where(
                            iot < H, es, 0.0)
                return carry2

            lax.fori_loop(0, _CH // 16, grp, 0)

        if ph < H:
            gstart(0, 0)
            gstart(1, 1)

            def duo(gg, carry):
                g0 = 2 * gg
                gwait(0)
                process(0, g0)
                sstart(0, g0)
                gwait(1)
                process(1, g0 + 1)
                swait(0)
                gstart(0, g0 + 2)
                sstart(1, g0 + 1)
                swait(1)
                gstart(1, g0 + 3)
                return carry

            lax.fori_loop(0, nch // 2 - 1, duo, 0)
            g0 = nch - 2
            gwait(0)
            process(0, g0)
            sstart(0, g0)
            gwait(1)
            process(1, g0 + 1)
            swait(0)
            sstart(1, g0 + 1)
            swait(1)
        else:
            def chunk(g, carry):
                process(0, g)
                pltpu.sync_copy(rows_v.at[0], out_sh.at[dst_v.at[g]],
                                add=True)
                return carry
            lax.fori_loop(0, nch, chunk, 0)

        plsc.subcore_barrier()
        pltpu.sync_copy(out_sh.at[pl.ds(r0, rows_per)],
                        out_hbm.at[c, ph, pl.ds(r0, rows_per)])


def _edge_pass1(xw_heads, ad_table, src3, dst3, e_real):
    nch = src3.shape[1]
    n = _NP
    xw_heads = jnp.pad(xw_heads, ((0, 0), (0, n - xw_heads.shape[1]), (0, 0)))
    ad_flat = jnp.pad(ad_table,
                      ((0, n - ad_table.shape[0]), (0, 0))).reshape(-1)
    mesh = plsc.VectorSubcoreMesh(core_axis_name="c", subcore_axis_name="s",
                                  num_cores=_NC, num_subcores=_NS)
    body = functools.partial(_edge_body1, e_real, nch, n)
    f = pl.kernel(
        body,
        out_type=jax.ShapeDtypeStruct((_NC, _H1 + 1, n, 16), jnp.float32),
        mesh=mesh,
        compiler_params=pltpu.CompilerParams(needs_layout_passes=False,
                                             use_tc_tiling_on_sc=False),
        scratch_types=[
            pltpu.VMEM((n * 2 * _H1,), jnp.float32),
            pltpu.VMEM((nch, _CH), jnp.int32),
            pltpu.VMEM((nch, _CH), jnp.int32),
            pltpu.VMEM((2, _CH, 16), jnp.float32),
            pltpu.VMEM((_CH, 16), jnp.float32),
            pltpu.SemaphoreType.DMA,
            pltpu.SemaphoreType.DMA,
            pltpu.SemaphoreType.DMA,
            pltpu.SemaphoreType.DMA,
            pltpu.VMEM_SHARED((n, 16), jnp.float32),
        ],
    )
    return f(xw_heads, ad_flat, src3, dst3)


def _edge_body2(e_real, nch, n,
                xw_hbm, ad_hbm, src_hbm, dst_hbm, out_hbm,
                ad_v, src_v, dst_v, rows_v, zer_v,
                sem0, sem1, ssem0, ssem1, out_sh):
    H = _H2
    c = lax.axis_index("c")
    s = lax.axis_index("s")
    wid = s * _NC + c
    rows_per = n // _NS
    r0 = s * rows_per
    aw = 2 * H
    sems = (sem0, sem1)
    ssems = (ssem0, ssem1)

    pltpu.sync_copy(ad_hbm, ad_v)
    pltpu.sync_copy(src_hbm.at[wid], src_v)
    pltpu.sync_copy(dst_hbm.at[wid], dst_v)

    def zrow(r, carry):
        zer_v[r, pl.ds(0, 16)] = jnp.zeros((16,), jnp.float32)
        return carry
    lax.fori_loop(0, _CH, zrow, 0)
    for j in range(rows_per // _CH):
        pltpu.sync_copy(zer_v, out_sh.at[pl.ds(r0 + j * _CH, _CH)])
    plsc.subcore_barrier()
    iot = lax.iota(jnp.int32, 16)

    def gstart(b, g):
        pltpu.async_copy(xw_hbm.at[src_v.at[g]], rows_v.at[b], sems[b])

    def gwait(b):
        pltpu.make_async_copy(xw_hbm.at[pl.ds(0, _CH)],
                              rows_v.at[b], sems[b]).wait()

    def sstart(b, g):
        pltpu.async_copy(rows_v.at[b], out_sh.at[dst_v.at[g]],
                         ssems[b], add=True)

    def swait(b):
        pltpu.make_async_copy(rows_v.at[b], out_sh.at[pl.ds(0, _CH)],
                              ssems[b]).wait()

    def process(b, g):
        base = (wid * nch + g) * _CH

        def grp(i, carry2):
            sv = src_v[g, pl.ds(i * 16, 16)]
            dv = dst_v[g, pl.ds(i * 16, 16)]
            valid = (base + i * 16 + iot) < e_real
            e = _logits(ad_v, sv * aw, dv * aw, H, 0, valid)
            for k in range(16):
                row = i * 16 + k
                v = rows_v[b, row, pl.ds(0, 16)]
                rows_v[b, row, pl.ds(0, 16)] = v * _splat(e, k)
            return carry2

        lax.fori_loop(0, _CH // 16, grp, 0)

    gstart(0, 0)
    gstart(1, 1)

    def duo(gg, carry):
        g0 = 2 * gg
        gwait(0)
        process(0, g0)
        sstart(0, g0)
        gwait(1)
        process(1, g0 + 1)
        swait(0)
        gstart(0, g0 + 2)
        sstart(1, g0 + 1)
        swait(1)
        gstart(1, g0 + 3)
        return carry

    lax.fori_loop(0, nch // 2 - 1, duo, 0)
    g0 = nch - 2
    gwait(0)
    process(0, g0)
    sstart(0, g0)
    gwait(1)
    process(1, g0 + 1)
    swait(0)
    sstart(1, g0 + 1)
    swait(1)

    plsc.subcore_barrier()
    pltpu.sync_copy(out_sh.at[pl.ds(r0, rows_per)],
                    out_hbm.at[c, pl.ds(r0, rows_per)])


def _edge_pass2(xw_table, ad_table, src3, dst3, e_real):
    nch = src3.shape[1]
    n = _NP
    xw_table = jnp.pad(xw_table, ((0, n - xw_table.shape[0]), (0, 0)))
    ad_flat = jnp.pad(ad_table,
                      ((0, n - ad_table.shape[0]), (0, 0))).reshape(-1)
    mesh = plsc.VectorSubcoreMesh(core_axis_name="c", subcore_axis_name="s",
                                  num_cores=_NC, num_subcores=_NS)
    body = functools.partial(_edge_body2, e_real, nch, n)
    f = pl.kernel(
        body,
        out_type=jax.ShapeDtypeStruct((_NC, n, 16), jnp.float32),
        mesh=mesh,
        compiler_params=pltpu.CompilerParams(needs_layout_passes=False,
                                             use_tc_tiling_on_sc=False),
        scratch_types=[
            pltpu.VMEM((n * 2 * _H2,), jnp.float32),
            pltpu.VMEM((nch, _CH), jnp.int32),
            pltpu.VMEM((nch, _CH), jnp.int32),
            pltpu.VMEM((2, _CH, 16), jnp.float32),
            pltpu.VMEM((_CH, 16), jnp.float32),
            pltpu.SemaphoreType.DMA,
            pltpu.SemaphoreType.DMA,
            pltpu.SemaphoreType.DMA,
            pltpu.SemaphoreType.DMA,
            pltpu.VMEM_SHARED((n, 16), jnp.float32),
        ],
    )
    return f(xw_table, ad_flat, src3, dst3)


def _att_mat(att, H, C):
    m = jnp.zeros((H * C, H), jnp.float32)
    rows = jnp.arange(H * C)
    return m.at[rows, rows // C].set(att.reshape(-1))


def kernel(x, edge_index, W1, att_src1, att_dst1, b1, W2, att_src2, att_dst2, b2):
    n = _N
    hc1 = _H1 * _C1
    loop = jnp.arange(n, dtype=edge_index.dtype)
    src = jnp.concatenate([edge_index[0], loop])
    dst = jnp.concatenate([edge_index[1], loop])
    e_real = src.shape[0]
    nch = -(-e_real // (_NW * _CH))
    nch += nch % 2
    ep = _NW * _CH * nch
    src3 = jnp.pad(src, (0, ep - e_real)).reshape(_NW, nch, _CH)
    dst3 = jnp.pad(dst, (0, ep - e_real)).reshape(_NW, nch, _CH)

    W1p = jnp.concatenate(
        [W1, W1 @ _att_mat(att_src1, _H1, _C1),
         W1 @ _att_mat(att_dst1, _H1, _C1)], axis=1)
    z1 = _matmul(x, W1p, bm=400)
    xw_heads = z1[:, :hc1].reshape(n, _H1, _C1).transpose(1, 0, 2)
    ad_tab1 = z1[:, hc1: hc1 + 2 * _H1]
    p1 = _edge_pass1(xw_heads, ad_tab1, src3, dst3, e_real)[:, :, :n]
    num1 = (p1[0, :_H1] + p1[1, :_H1]).transpose(1, 0, 2).reshape(n, hc1)
    den1 = jnp.repeat((p1[0, _H1] + p1[1, _H1])[:, :_H1],
                      _C1, axis=1)

    W2p = jnp.concatenate(
        [W2, (W2 @ att_src2.reshape(-1))[:, None],
         (W2 @ att_dst2.reshape(-1))[:, None]], axis=1)
    z2 = _combine1(num1, den1, b1.reshape(1, -1), W2p, bm=2000)
    xw_tab2 = jnp.concatenate(
        [z2[:, :7], jnp.ones((n, 1), jnp.float32),
         jnp.zeros((n, 8), jnp.float32)], axis=1)
    ad_tab2 = z2[:, 6:8]
    p2 = _edge_pass2(xw_tab2, ad_tab2, src3, dst3, e_real)[:, :n]
    num2 = p2[0] + p2[1]
    den2 = jnp.repeat(num2[:, 7:8], 16, axis=1)

    b2p = jnp.pad(b2, (0, 10)).reshape(1, 16)
    return _final(num2, den2, b2p, bm=2000)


# speedup vs baseline: 1.1101x; 1.1101x over previous
import functools

import jax
import jax.numpy as jnp
from jax import lax
from jax.experimental import pallas as pl
from jax.experimental.pallas import tpu as pltpu
from jax.experimental.pallas import tpu_sc as plsc

_N = 10000
_F_IN = 3703
_H1, _C1 = 4, 16
_H2, _C2 = 1, 6

_NC, _NS = 2, 16
_NW = _NC * _NS
_CH = 128
_NP = 10240


def _mm_body(a_ref, b_ref, o_ref):
    o_ref[...] = jnp.dot(a_ref[...], b_ref[...],
                         preferred_element_type=jnp.float32)


def _matmul(a, b, bm):
    m, k = a.shape
    _, n = b.shape
    return pl.pallas_call(
        _mm_body,
        grid=(m // bm,),
        in_specs=[
            pl.BlockSpec((bm, k), lambda i: (i, 0)),
            pl.BlockSpec((k, n), lambda i: (0, 0)),
        ],
        out_specs=pl.BlockSpec((bm, n), lambda i: (i, 0)),
        out_shape=jax.ShapeDtypeStruct((m, n), jnp.float32),
    )(a, b)


def _comb_body(num_ref, den_ref, b_ref, w_ref, o_ref):
    h = jnp.maximum(num_ref[...] / den_ref[...] + b_ref[...], 0.0)
    o_ref[...] = jnp.dot(h, w_ref[...], preferred_element_type=jnp.float32)


def _combine1(num, den, b, w, bm):
    m, k = num.shape
    n = w.shape[1]
    return pl.pallas_call(
        _comb_body,
        grid=(m // bm,),
        in_specs=[
            pl.BlockSpec((bm, k), lambda i: (i, 0)),
            pl.BlockSpec((bm, k), lambda i: (i, 0)),
            pl.BlockSpec((1, k), lambda i: (0, 0)),
            pl.BlockSpec((k, n), lambda i: (0, 0)),
        ],
        out_specs=pl.BlockSpec((bm, n), lambda i: (i, 0)),
        out_shape=jax.ShapeDtypeStruct((m, n), jnp.float32),
    )(num, den, b, w)


def _final_body(num_ref, den_ref, b_ref, o_ref):
    z = num_ref[...] / den_ref[...] + b_ref[...]
    col = lax.broadcasted_iota(jnp.int32, z.shape, 1)
    zm = jnp.where(col < _H2 * _C2, z, -jnp.inf)
    m = jnp.max(zm, axis=1, keepdims=True)
    s = jnp.sum(jnp.where(col < _H2 * _C2, jnp.exp(z - m), 0.0),
                axis=1, keepdims=True)
    o_ref[...] = (z - m - jnp.log(s))[:, : _H2 * _C2]


def _final(num, den, b, bm):
    m, k = num.shape
    return pl.pallas_call(
        _final_body,
        grid=(m // bm,),
        in_specs=[
            pl.BlockSpec((bm, k), lambda i: (i, 0)),
            pl.BlockSpec((bm, k), lambda i: (i, 0)),
            pl.BlockSpec((1, k), lambda i: (0, 0)),
        ],
        out_specs=pl.BlockSpec((bm, _H2 * _C2), lambda i: (i, 0)),
        out_shape=jax.ShapeDtypeStruct((m, _H2 * _C2), jnp.float32),
    )(num, den, b)


def _splat(v, j):
    return v.at[jnp.full((16,), j, jnp.int32)].get(mode="promise_in_bounds")


def _logits(ad_v, sva, dva, H, h, valid):
    a_s = plsc.load_gather(ad_v, [sva + h])
    a_d = plsc.load_gather(ad_v, [dva + (H + h)])
    al = a_s + a_d
    al = jnp.where(al >= 0, al, al * 0.2)
    return jnp.where(valid, jnp.exp(al), 0.0)


def _edge_body1(e_real, nch, n,
                xw_hbm, ad_hbm, src_hbm, dst_hbm, out_hbm,
                ad_v, src_v, dst_v, rows_v, zer_v, sem0, sem1, out_sh):
    H = _H1
    c = lax.axis_index("c")
    s = lax.axis_index("s")
    wid = s * _NC + c
    rows_per = n // _NS
    r0 = s * rows_per
    aw = 2 * H
    sems = (sem0, sem1)

    pltpu.sync_copy(ad_hbm, ad_v)
    pltpu.sync_copy(src_hbm.at[wid], src_v)
    pltpu.sync_copy(dst_hbm.at[wid], dst_v)

    def zrow(r, carry):
        zer_v[r, pl.ds(0, 16)] = jnp.zeros((16,), jnp.float32)
        return carry
    lax.fori_loop(0, _CH, zrow, 0)
    iot = lax.iota(jnp.int32, 16)

    for ph in range(H + 1):
        for j in range(rows_per // _CH):
            pltpu.sync_copy(zer_v, out_sh.at[pl.ds(r0 + j * _CH, _CH)])
        plsc.subcore_barrier()

        def gstart(b, g):
            pltpu.async_copy(xw_hbm.at[ph].at[src_v.at[g]],
                             rows_v.at[b], sems[b])

        def gwait(b):
            pltpu.make_async_copy(xw_hbm.at[ph].at[pl.ds(0, _CH)],
                                  rows_v.at[b], sems[b]).wait()

        def process(b, g):
            base = (wid * nch + g) * _CH

            def grp(i, carry2):
                sv = src_v[g, pl.ds(i * 16, 16)]
                dv = dst_v[g, pl.ds(i * 16, 16)]
                valid = (base + i * 16 + iot) < e_real
                sva = sv * aw
                dva = dv * aw
                if ph < H:
                    e = _logits(ad_v, sva, dva, H, ph, valid)
                    for k in range(16):
                        row = i * 16 + k
                        v = rows_v[b, row, pl.ds(0, 16)]
                        rows_v[b, row, pl.ds(0, 16)] = v * _splat(e, k)
                else:
                    eh = [_logits(ad_v, sva, dva, H, h, valid)
                          for h in range(H)]
                    for k in range(16):
                        row = i * 16 + k
                        es = _splat(eh[0], k)
                        for h in range(1, H):
                            es = jnp.where(iot == h, _splat(eh[h], k), es)
                        rows_v[b, row, pl.ds(0, 16)] = jnp.where(
                            iot < H, es, 0.0)
                return carry2

            lax.fori_loop(0, _CH // 16, grp, 0)
            pltpu.sync_copy(rows_v.at[b], out_sh.at[dst_v.at[g]], add=True)

        if ph < H:
            gstart(0, 0)

            def duo(gg, carry):
                g0 = 2 * gg
                gstart(1, g0 + 1)
                gwait(0)
                process(0, g0)
                gstart(0, g0 + 2)
                gwait(1)
                process(1, g0 + 1)
                return carry

            lax.fori_loop(0, nch // 2 - 1, duo, 0)
            g0 = nch - 2
            gstart(1, g0 + 1)
            gwait(0)
            process(0, g0)
            gwait(1)
            process(1, g0 + 1)
        else:
            def chunk(g, carry):
                process(0, g)
                return carry
            lax.fori_loop(0, nch, chunk, 0)

        plsc.subcore_barrier()
        pltpu.sync_copy(out_sh.at[pl.ds(r0, rows_per)],
                        out_hbm.at[c, ph, pl.ds(r0, rows_per)])


def _edge_pass1(xw_heads, ad_table, src3, dst3, e_real):
    nch = src3.shape[1]
    n = _NP
    xw_heads = jnp.pad(xw_heads, ((0, 0), (0, n - xw_heads.shape[1]), (0, 0)))
    ad_flat = jnp.pad(ad_table,
                      ((0, n - ad_table.shape[0]), (0, 0))).reshape(-1)
    mesh = plsc.VectorSubcoreMesh(core_axis_name="c", subcore_axis_name="s",
                                  num_cores=_NC, num_subcores=_NS)
    body = functools.partial(_edge_body1, e_real, nch, n)
    f = pl.kernel(
        body,
        out_type=jax.ShapeDtypeStruct((_NC, _H1 + 1, n, 16), jnp.float32),
        mesh=mesh,
        compiler_params=pltpu.CompilerParams(needs_layout_passes=False,
                                             use_tc_tiling_on_sc=False),
        scratch_types=[
            pltpu.VMEM((n * 2 * _H1,), jnp.float32),
            pltpu.VMEM((nch, _CH), jnp.int32),
            pltpu.VMEM((nch, _CH), jnp.int32),
            pltpu.VMEM((2, _CH, 16), jnp.float32),
            pltpu.VMEM((_CH, 16), jnp.float32),
            pltpu.SemaphoreType.DMA,
            pltpu.SemaphoreType.DMA,
            pltpu.VMEM_SHARED((n, 16), jnp.float32),
        ],
    )
    return f(xw_heads, ad_flat, src3, dst3)


def _edge_body2(e_real, nch, n,
                xw_hbm, ad_hbm, src_hbm, dst_hbm, out_hbm,
                ad_v, src_v, dst_v, rows_v, zer_v, sem0, sem1, out_sh):
    H = _H2
    c = lax.axis_index("c")
    s = lax.axis_index("s")
    wid = s * _NC + c
    rows_per = n // _NS
    r0 = s * rows_per
    aw = 2 * H
    sems = (sem0, sem1)

    pltpu.sync_copy(ad_hbm, ad_v)
    pltpu.sync_copy(src_hbm.at[wid], src_v)
    pltpu.sync_copy(dst_hbm.at[wid], dst_v)

    def zrow(r, carry):
        zer_v[r, pl.ds(0, 16)] = jnp.zeros((16,), jnp.float32)
        return carry
    lax.fori_loop(0, _CH, zrow, 0)
    for j in range(rows_per // _CH):
        pltpu.sync_copy(zer_v, out_sh.at[pl.ds(r0 + j * _CH, _CH)])
    plsc.subcore_barrier()
    iot = lax.iota(jnp.int32, 16)

    def gstart(b, g):
        pltpu.async_copy(xw_hbm.at[src_v.at[g]], rows_v.at[b], sems[b])

    def gwait(b):
        pltpu.make_async_copy(xw_hbm.at[pl.ds(0, _CH)],
                              rows_v.at[b], sems[b]).wait()

    def process(b, g):
        base = (wid * nch + g) * _CH

        def grp(i, carry2):
            sv = src_v[g, pl.ds(i * 16, 16)]
            dv = dst_v[g, pl.ds(i * 16, 16)]
            valid = (base + i * 16 + iot) < e_real
            e = _logits(ad_v, sv * aw, dv * aw, H, 0, valid)
            for k in range(16):
                row = i * 16 + k
                v = rows_v[b, row, pl.ds(0, 16)]
                rows_v[b, row, pl.ds(0, 16)] = v * _splat(e, k)
            return carry2

        lax.fori_loop(0, _CH // 16, grp, 0)
        pltpu.sync_copy(rows_v.at[b], out_sh.at[dst_v.at[g]], add=True)

    gstart(0, 0)

    def duo(gg, carry):
        g0 = 2 * gg
        gstart(1, g0 + 1)
        gwait(0)
        process(0, g0)
        gstart(0, g0 + 2)
        gwait(1)
        process(1, g0 + 1)
        return carry

    lax.fori_loop(0, nch // 2 - 1, duo, 0)
    g0 = nch - 2
    gstart(1, g0 + 1)
    gwait(0)
    process(0, g0)
    gwait(1)
    process(1, g0 + 1)

    plsc.subcore_barrier()
    pltpu.sync_copy(out_sh.at[pl.ds(r0, rows_per)],
                    out_hbm.at[c, pl.ds(r0, rows_per)])


def _edge_pass2(xw_table, ad_table, src3, dst3, e_real):
    nch = src3.shape[1]
    n = _NP
    xw_table = jnp.pad(xw_table, ((0, n - xw_table.shape[0]), (0, 0)))
    ad_flat = jnp.pad(ad_table,
                      ((0, n - ad_table.shape[0]), (0, 0))).reshape(-1)
    mesh = plsc.VectorSubcoreMesh(core_axis_name="c", subcore_axis_name="s",
                                  num_cores=_NC, num_subcores=_NS)
    body = functools.partial(_edge_body2, e_real, nch, n)
    f = pl.kernel(
        body,
        out_type=jax.ShapeDtypeStruct((_NC, n, 16), jnp.float32),
        mesh=mesh,
        compiler_params=pltpu.CompilerParams(needs_layout_passes=False,
                                             use_tc_tiling_on_sc=False),
        scratch_types=[
            pltpu.VMEM((n * 2 * _H2,), jnp.float32),
            pltpu.VMEM((nch, _CH), jnp.int32),
            pltpu.VMEM((nch, _CH), jnp.int32),
            pltpu.VMEM((2, _CH, 16), jnp.float32),
            pltpu.VMEM((_CH, 16), jnp.float32),
            pltpu.SemaphoreType.DMA,
            pltpu.SemaphoreType.DMA,
            pltpu.VMEM_SHARED((n, 16), jnp.float32),
        ],
    )
    return f(xw_table, ad_flat, src3, dst3)


def _att_mat(att, H, C):
    m = jnp.zeros((H * C, H), jnp.float32)
    rows = jnp.arange(H * C)
    return m.at[rows, rows // C].set(att.reshape(-1))


def kernel(x, edge_index, W1, att_src1, att_dst1, b1, W2, att_src2, att_dst2, b2):
    n = _N
    hc1 = _H1 * _C1
    loop = jnp.arange(n, dtype=edge_index.dtype)
    src = jnp.concatenate([edge_index[0], loop])
    dst = jnp.concatenate([edge_index[1], loop])
    e_real = src.shape[0]
    nch = -(-e_real // (_NW * _CH))
    nch += nch % 2
    ep = _NW * _CH * nch
    src3 = jnp.pad(src, (0, ep - e_real)).reshape(_NW, nch, _CH)
    dst3 = jnp.pad(dst, (0, ep - e_real)).reshape(_NW, nch, _CH)

    W1p = jnp.concatenate(
        [W1, W1 @ _att_mat(att_src1, _H1, _C1),
         W1 @ _att_mat(att_dst1, _H1, _C1)], axis=1)
    z1 = _matmul(x, W1p, bm=400)
    xw_heads = z1[:, :hc1].reshape(n, _H1, _C1).transpose(1, 0, 2)
    ad_tab1 = z1[:, hc1: hc1 + 2 * _H1]
    p1 = _edge_pass1(xw_heads, ad_tab1, src3, dst3, e_real)[:, :, :n]
    num1 = (p1[0, :_H1] + p1[1, :_H1]).transpose(1, 0, 2).reshape(n, hc1)
    den1 = jnp.repeat((p1[0, _H1] + p1[1, _H1])[:, :_H1],
                      _C1, axis=1)

    W2p = jnp.concatenate(
        [W2, (W2 @ att_src2.reshape(-1))[:, None],
         (W2 @ att_dst2.reshape(-1))[:, None]], axis=1)
    z2 = _combine1(num1, den1, b1.reshape(1, -1), W2p, bm=2000)
    xw_tab2 = jnp.concatenate(
        [z2[:, :7], jnp.ones((n, 1), jnp.float32),
         jnp.zeros((n, 8), jnp.float32)], axis=1)
    ad_tab2 = z2[:, 6:8]
    p2 = _edge_pass2(xw_tab2, ad_tab2, src3, dst3, e_real)[:, :n]
    num2 = p2[0] + p2[1]
    den2 = jnp.repeat(num2[:, 7:8], 16, axis=1)

    b2p = jnp.pad(b2, (0, 10)).reshape(1, 16)
    return _final(num2, den2, b2p, bm=2000)
